# baseline (speedup 1.0000x reference)
import functools

import jax
import jax.numpy as jnp
from jax import lax
from jax.experimental import pallas as pl
from jax.experimental.pallas import tpu as pltpu
from jax.experimental.pallas import tpu_sc as plsc

_NUM_CORES = 2
_NUM_SUBCORES = 16
_NUM_WORKERS = _NUM_CORES * _NUM_SUBCORES
_CHUNK_ROWS = 128


def _broadcast_rows(emb, batch):
    n_rows, dim = emb.shape
    rows_per_w = n_rows // _NUM_WORKERS
    n_chunks = rows_per_w // _CHUNK_ROWS

    mesh = plsc.VectorSubcoreMesh(core_axis_name="c", subcore_axis_name="s")

    @functools.partial(
        pl.kernel,
        mesh=mesh,
        out_type=jax.ShapeDtypeStruct((batch, n_rows, dim), jnp.float32),
        scratch_types=[
            pltpu.VMEM((_CHUNK_ROWS, dim), jnp.float32),
            pltpu.SemaphoreType.DMA,
        ],
    )
    def k(emb_hbm, out_hbm, buf, sem):
        wid = lax.axis_index("s") * _NUM_CORES + lax.axis_index("c")
        base = wid * rows_per_w
        for i in range(n_chunks):
            r0 = base + i * _CHUNK_ROWS
            pltpu.sync_copy(emb_hbm.at[pl.ds(r0, _CHUNK_ROWS), :], buf)
            copies = [
                pltpu.async_copy(
                    buf, out_hbm.at[b, pl.ds(r0, _CHUNK_ROWS), :], sem
                )
                for b in range(batch)
            ]
            for c in copies:
                c.wait()

    return k(emb)


def kernel(x, emb):
    batch = x.shape[0]
    return _broadcast_rows(emb, batch)
